# baseline (speedup 1.0000x reference)
import jax
import jax.numpy as jnp
from jax import lax
from jax.experimental import pallas as pl
from jax.experimental.pallas import tpu as pltpu

N = 10000
E = 320000
D = 128
H = 256
ED = 16
G = 64
OUT = 2


def _edge_body(ea_ref, we_ref, be_ref, o_ref):
    o_ref[...] = lax.dot_general(
        ea_ref[...], we_ref[...], (((1,), (0,)), ((), ())),
        preferred_element_type=jnp.float32) + be_ref[...]


def _edge_transform(edge_attr, We, be):
    BE = 2000
    din = We.shape[1]
    return pl.pallas_call(
        _edge_body,
        grid=(E // BE,),
        in_specs=[
            pl.BlockSpec((BE, ED), lambda i: (i, 0)),
            pl.BlockSpec((ED, din), lambda i: (0, 0)),
            pl.BlockSpec((1, din), lambda i: (0, 0)),
        ],
        out_specs=pl.BlockSpec((BE, din), lambda i: (i, 0)),
        out_shape=jax.ShapeDtypeStruct((E, din), jnp.float32),
    )(edge_attr, We, be.reshape(1, din))


def _pre_body(x_ref, p_ref, wa_ref, ba_ref, t_ref, st_ref):
    i = pl.program_id(0)
    xb = x_ref[...] + p_ref[...]
    t = lax.dot_general(xb, wa_ref[...], (((1,), (0,)), ((), ())),
                        preferred_element_type=jnp.float32) + ba_ref[...]
    t_ref[...] = t

    @pl.when(i == 0)
    def _init():
        st_ref[...] = jnp.zeros_like(st_ref)

    s = jnp.sum(t, axis=0, keepdims=True)
    q = jnp.sum(t * t, axis=0, keepdims=True)
    st_ref[...] += jnp.concatenate([s, q], axis=0)


def _pre_matmul(x, aggr, Wa, ba):
    BR = 1000
    din = Wa.shape[0]
    return pl.pallas_call(
        _pre_body,
        grid=(N // BR,),
        in_specs=[
            pl.BlockSpec((BR, din), lambda i: (i, 0)),
            pl.BlockSpec((BR, din), lambda i: (i, 0)),
            pl.BlockSpec((din, H), lambda i: (0, 0)),
            pl.BlockSpec((1, H), lambda i: (0, 0)),
        ],
        out_specs=(pl.BlockSpec((BR, H), lambda i: (i, 0)),
                   pl.BlockSpec((2, H), lambda i: (0, 0))),
        out_shape=(jax.ShapeDtypeStruct((N, H), jnp.float32),
                   jax.ShapeDtypeStruct((2, H), jnp.float32)),
    )(x, aggr, Wa, ba.reshape(1, H))


def _post_body(t_ref, a_ref, c_ref, wb_ref, bb_ref, o_ref):
    z = jnp.maximum(t_ref[...] * a_ref[...] + c_ref[...], 0.0)
    u = lax.dot_general(z, wb_ref[...], (((1,), (0,)), ((), ())),
                        preferred_element_type=jnp.float32) + bb_ref[...]
    o_ref[...] = jnp.maximum(u, 0.0)


def _post_matmul(t, a, c, Wb, bb):
    BR = 1000
    return pl.pallas_call(
        _post_body,
        grid=(N // BR,),
        in_specs=[
            pl.BlockSpec((BR, H), lambda i: (i, 0)),
            pl.BlockSpec((1, H), lambda i: (0, 0)),
            pl.BlockSpec((1, H), lambda i: (0, 0)),
            pl.BlockSpec((H, H), lambda i: (0, 0)),
            pl.BlockSpec((1, H), lambda i: (0, 0)),
        ],
        out_specs=pl.BlockSpec((BR, H), lambda i: (i, 0)),
        out_shape=jax.ShapeDtypeStruct((N, H), jnp.float32),
    )(t, a.reshape(1, H), c.reshape(1, H), Wb, bb.reshape(1, H))


def _pool_body(oh_ref, h_ref, wm1_ref, bm1_ref, wm2_ref, bm2_ref,
               out_ref, sums_ref, counts_ref):
    i = pl.program_id(0)
    nblk = pl.num_programs(0)
    oh = oh_ref[...]

    @pl.when(i == 0)
    def _init():
        sums_ref[...] = jnp.zeros_like(sums_ref)
        counts_ref[...] = jnp.zeros_like(counts_ref)

    sums_ref[...] += lax.dot_general(oh, h_ref[...], (((0,), (0,)), ((), ())),
                                     preferred_element_type=jnp.float32)
    counts_ref[...] += jnp.sum(oh, axis=0, keepdims=True)

    @pl.when(i == nblk - 1)
    def _final():
        pooled = sums_ref[...] / jnp.maximum(counts_ref[...], 1.0).T
        z = jnp.maximum(
            lax.dot_general(pooled, wm1_ref[...], (((1,), (0,)), ((), ())),
                            preferred_element_type=jnp.float32)
            + bm1_ref[...], 0.0)
        z = lax.dot_general(z, wm2_ref[...], (((1,), (0,)), ((), ())),
                            preferred_element_type=jnp.float32) + bm2_ref[...]
        z = z - jnp.max(z, axis=1, keepdims=True)
        ez = jnp.exp(z)
        out_ref[...] = ez / jnp.sum(ez, axis=1, keepdims=True)


def _pool_mlp(h, onehot, Wm1, bm1, Wm2, bm2):
    BR = 1000
    return pl.pallas_call(
        _pool_body,
        grid=(N // BR,),
        in_specs=[
            pl.BlockSpec((BR, G), lambda i: (i, 0)),
            pl.BlockSpec((BR, H), lambda i: (i, 0)),
            pl.BlockSpec((H, H), lambda i: (0, 0)),
            pl.BlockSpec((1, H), lambda i: (0, 0)),
            pl.BlockSpec((H, OUT), lambda i: (0, 0)),
            pl.BlockSpec((1, OUT), lambda i: (0, 0)),
        ],
        out_specs=pl.BlockSpec((G, OUT), lambda i: (0, 0)),
        out_shape=jax.ShapeDtypeStruct((G, OUT), jnp.float32),
        scratch_shapes=[
            pltpu.VMEM((G, H), jnp.float32),
            pltpu.VMEM((1, G), jnp.float32),
        ],
    )(onehot, h, Wm1, bm1.reshape(1, H), Wm2, bm2.reshape(1, OUT))


def _bn_coeffs(st, g, bt):
    mean = st[0] / N
    var = st[1] / N - mean * mean
    inv = g * lax.rsqrt(var + 1e-5)
    return inv, bt - mean * inv


def kernel(x, edge_index, edge_attr, batch,
           We1, be1, Wa1, ba1, g1, bt1, Wb1, bb1,
           We2, be2, Wa2, ba2, g2, bt2, Wb2, bb2,
           We3, be3, Wa3, ba3, g3, bt3, Wb3, bb3,
           Wm1, bm1, Wm2, bm2):
    src = edge_index[0]
    dst = edge_index[1]
    h = x
    for (We, be, Wa, ba, g, bt, Wb, bb) in (
            (We1, be1, Wa1, ba1, g1, bt1, Wb1, bb1),
            (We2, be2, Wa2, ba2, g2, bt2, Wb2, bb2),
            (We3, be3, Wa3, ba3, g3, bt3, Wb3, bb3)):
        e = _edge_transform(edge_attr, We, be)
        m = jax.nn.relu(h[src] + e)
        aggr = jax.ops.segment_sum(m, dst, num_segments=N)
        t, st = _pre_matmul(h, aggr, Wa, ba)
        a, c = _bn_coeffs(st, g, bt)
        h = _post_matmul(t, a, c, Wb, bb)
    onehot = jax.nn.one_hot(batch, G, dtype=jnp.float32)
    return _pool_mlp(h, onehot, Wm1, bm1, Wm2, bm2)


# speedup vs baseline: 1.0771x; 1.0771x over previous
import jax
import jax.numpy as jnp
from jax import lax
from jax.experimental import pallas as pl
from jax.experimental.pallas import tpu as pltpu

N = 10000
E = 320000
D = 128
H = 256
ED = 16
G = 64
OUT = 2


def _edge_body(ea_ref, we_ref, be_ref, o_ref):
    o_ref[...] = lax.dot_general(
        ea_ref[...], we_ref[...], (((1,), (0,)), ((), ())),
        preferred_element_type=jnp.float32) + be_ref[...]


def _edge_transform(edge_attr, We, be):
    BE = 2000
    din = We.shape[1]
    return pl.pallas_call(
        _edge_body,
        grid=(E // BE,),
        in_specs=[
            pl.BlockSpec((BE, ED), lambda i: (i, 0)),
            pl.BlockSpec((ED, din), lambda i: (0, 0)),
            pl.BlockSpec((1, din), lambda i: (0, 0)),
        ],
        out_specs=pl.BlockSpec((BE, din), lambda i: (i, 0)),
        out_shape=jax.ShapeDtypeStruct((E, din), jnp.float32),
    )(edge_attr, We, be.reshape(1, din))


def _pre_body(x_ref, p_ref, wa_ref, ba_ref, t_ref, st_ref):
    i = pl.program_id(0)
    xb = x_ref[...] + p_ref[...]
    t = lax.dot_general(xb, wa_ref[...], (((1,), (0,)), ((), ())),
                        preferred_element_type=jnp.float32) + ba_ref[...]
    t_ref[...] = t

    @pl.when(i == 0)
    def _init():
        st_ref[...] = jnp.zeros_like(st_ref)

    s = jnp.sum(t, axis=0, keepdims=True)
    q = jnp.sum(t * t, axis=0, keepdims=True)
    st_ref[...] += jnp.concatenate([s, q], axis=0)


def _pre_matmul(x, aggr, Wa, ba):
    BR = 1000
    din = Wa.shape[0]
    return pl.pallas_call(
        _pre_body,
        grid=(N // BR,),
        in_specs=[
            pl.BlockSpec((BR, din), lambda i: (i, 0)),
            pl.BlockSpec((BR, din), lambda i: (i, 0)),
            pl.BlockSpec((din, H), lambda i: (0, 0)),
            pl.BlockSpec((1, H), lambda i: (0, 0)),
        ],
        out_specs=(pl.BlockSpec((BR, H), lambda i: (i, 0)),
                   pl.BlockSpec((2, H), lambda i: (0, 0))),
        out_shape=(jax.ShapeDtypeStruct((N, H), jnp.float32),
                   jax.ShapeDtypeStruct((2, H), jnp.float32)),
    )(x, aggr, Wa, ba.reshape(1, H))


def _post_body(t_ref, a_ref, c_ref, wb_ref, bb_ref, o_ref):
    z = jnp.maximum(t_ref[...] * a_ref[...] + c_ref[...], 0.0)
    u = lax.dot_general(z, wb_ref[...], (((1,), (0,)), ((), ())),
                        preferred_element_type=jnp.float32) + bb_ref[...]
    o_ref[...] = jnp.maximum(u, 0.0)


def _post_matmul(t, a, c, Wb, bb):
    BR = 1000
    return pl.pallas_call(
        _post_body,
        grid=(N // BR,),
        in_specs=[
            pl.BlockSpec((BR, H), lambda i: (i, 0)),
            pl.BlockSpec((1, H), lambda i: (0, 0)),
            pl.BlockSpec((1, H), lambda i: (0, 0)),
            pl.BlockSpec((H, H), lambda i: (0, 0)),
            pl.BlockSpec((1, H), lambda i: (0, 0)),
        ],
        out_specs=pl.BlockSpec((BR, H), lambda i: (i, 0)),
        out_shape=jax.ShapeDtypeStruct((N, H), jnp.float32),
    )(t, a.reshape(1, H), c.reshape(1, H), Wb, bb.reshape(1, H))


def _pool_body(oh_ref, h_ref, wm1_ref, bm1_ref, wm2_ref, bm2_ref,
               out_ref, sums_ref, counts_ref):
    i = pl.program_id(0)
    nblk = pl.num_programs(0)
    oh = oh_ref[...]

    @pl.when(i == 0)
    def _init():
        sums_ref[...] = jnp.zeros_like(sums_ref)
        counts_ref[...] = jnp.zeros_like(counts_ref)

    sums_ref[...] += lax.dot_general(oh, h_ref[...], (((0,), (0,)), ((), ())),
                                     preferred_element_type=jnp.float32)
    counts_ref[...] += jnp.sum(oh, axis=0, keepdims=True)

    @pl.when(i == nblk - 1)
    def _final():
        pooled = sums_ref[...] / jnp.maximum(counts_ref[...], 1.0).T
        z = jnp.maximum(
            lax.dot_general(pooled, wm1_ref[...], (((1,), (0,)), ((), ())),
                            preferred_element_type=jnp.float32)
            + bm1_ref[...], 0.0)
        z = lax.dot_general(z, wm2_ref[...], (((1,), (0,)), ((), ())),
                            preferred_element_type=jnp.float32) + bm2_ref[...]
        z = z - jnp.max(z, axis=1, keepdims=True)
        ez = jnp.exp(z)
        out_ref[...] = ez / jnp.sum(ez, axis=1, keepdims=True)


def _pool_mlp(h, onehot, Wm1, bm1, Wm2, bm2):
    BR = 1000
    return pl.pallas_call(
        _pool_body,
        grid=(N // BR,),
        in_specs=[
            pl.BlockSpec((BR, G), lambda i: (i, 0)),
            pl.BlockSpec((BR, H), lambda i: (i, 0)),
            pl.BlockSpec((H, H), lambda i: (0, 0)),
            pl.BlockSpec((1, H), lambda i: (0, 0)),
            pl.BlockSpec((H, OUT), lambda i: (0, 0)),
            pl.BlockSpec((1, OUT), lambda i: (0, 0)),
        ],
        out_specs=pl.BlockSpec((G, OUT), lambda i: (0, 0)),
        out_shape=jax.ShapeDtypeStruct((G, OUT), jnp.float32),
        scratch_shapes=[
            pltpu.VMEM((G, H), jnp.float32),
            pltpu.VMEM((1, G), jnp.float32),
        ],
    )(onehot, h, Wm1, bm1.reshape(1, H), Wm2, bm2.reshape(1, OUT))


def _bn_coeffs(st, g, bt):
    mean = st[0] / N
    var = st[1] / N - mean * mean
    inv = g * lax.rsqrt(var + 1e-5)
    return inv, bt - mean * inv


def kernel(x, edge_index, edge_attr, batch,
           We1, be1, Wa1, ba1, g1, bt1, Wb1, bb1,
           We2, be2, Wa2, ba2, g2, bt2, Wb2, bb2,
           We3, be3, Wa3, ba3, g3, bt3, Wb3, bb3,
           Wm1, bm1, Wm2, bm2):
    src = edge_index[0]
    dst = edge_index[1]
    h = x
    for (We, be, Wa, ba, g, bt, Wb, bb) in (
            (We1, be1, Wa1, ba1, g1, bt1, Wb1, bb1),
            (We2, be2, Wa2, ba2, g2, bt2, Wb2, bb2),
            (We3, be3, Wa3, ba3, g3, bt3, Wb3, bb3)):
        m = jax.nn.relu(h[src] + (edge_attr @ We + be))
        aggr = jax.ops.segment_sum(m, dst, num_segments=N)
        t, st = _pre_matmul(h, aggr, Wa, ba)
        a, c = _bn_coeffs(st, g, bt)
        h = _post_matmul(t, a, c, Wb, bb)
    onehot = jax.nn.one_hot(batch, G, dtype=jnp.float32)
    return _pool_mlp(h, onehot, Wm1, bm1, Wm2, bm2)
